# baseline (speedup 1.0000x reference)
import functools

import jax
import jax.numpy as jnp
from jax import lax
from jax.experimental import pallas as pl
from jax.experimental.pallas import tpu as pltpu
from jax.experimental.pallas import tpu_sc as plsc

_BATCH = 4
_SEQ = 8192
_EMBED = 1024


_TC_BLK = 128


def _tc_add_body(in_ref, tab_ref, out_ref):
    out_ref[...] = in_ref[...] + tab_ref[...][None, :, :]


def _tc_add(inputs, pos_table):
    batch, seq_len, embed = inputs.shape
    return pl.pallas_call(
        _tc_add_body,
        grid=(seq_len // _TC_BLK,),
        in_specs=[
            pl.BlockSpec((batch, _TC_BLK, embed), lambda i: (0, i, 0)),
            pl.BlockSpec((_TC_BLK, embed), lambda i: (i, 0)),
        ],
        out_specs=pl.BlockSpec((batch, _TC_BLK, embed), lambda i: (0, i, 0)),
        out_shape=jax.ShapeDtypeStruct((batch, seq_len, embed), inputs.dtype),
    )(inputs, pos_table)


_SC_CORES = 2
_SC_SUBCORES = 16
_NW = _SC_CORES * _SC_SUBCORES
_T = 8
_U = 8


_GT = 16


def _sc_body_ga(seq_rows, in_hbm, tab_hbm, out_hbm,
                in_v, idx_v, s_ld0, s_ld1, s_ga, s_st0, s_st1):
    wid = lax.axis_index("s") * _SC_CORES + lax.axis_index("c")
    rows_per_w = seq_rows // _NW
    nchunks = rows_per_w // _GT
    nsteps = nchunks * 2
    base_row = wid * rows_per_w
    s_ld = (s_ld0, s_ld1)
    s_st = (s_st0, s_st1)

    def row0_of_step(t):
        return base_row + (t // 2) * _GT

    def start_load(t, d, i):
        b = 2 * d + i
        return pltpu.make_async_copy(
            in_hbm.at[b, pl.ds(row0_of_step(t), _GT), :], in_v.at[d, i], s_ld[d])

    def start_store(t, d, i):
        b = 2 * d + i
        return pltpu.make_async_copy(
            in_v.at[d, i], out_hbm.at[b, pl.ds(row0_of_step(t), _GT), :], s_st[d])

    def ga_descr(d, i):
        return pltpu.make_async_copy(tab_hbm.at[idx_v], in_v.at[d, i], s_ga)

    for i in range(2):
        start_load(0, 0, i).start()

    lanes = lax.broadcasted_iota(jnp.int32, (16,), 0)

    def body(k):
        for d in range(2):
            t = k + d
            for i in range(2):
                start_load(t, d, i).wait()
            if d == 0:
                idx_v[...] = row0_of_step(t) + lanes
            for i in range(2):
                pltpu.async_copy(tab_hbm.at[idx_v], in_v.at[d, i], s_ga,
                                 add=True)
            for i in range(2):
                ga_descr(d, i).wait()
            for i in range(2):
                start_store(t, d, i).start()
            if d == 1:
                for i in range(2):
                    start_store(t - 1, 0, i).wait()
            else:
                @pl.when(k > 0)
                def _():
                    for i in range(2):
                        start_store(t - 1, 1, i).wait()
            if d == 0:
                for i in range(2):
                    start_load(t + 1, 1, i).start()
            else:
                @pl.when(k < nsteps - 2)
                def _():
                    for i in range(2):
                        start_load(t + 1, 0, i).start()

    pl.loop(0, nsteps, step=2)(body)
    for i in range(2):
        start_store(nsteps - 1, 1, i).wait()


def _sc_add_ga(inputs, pos_table):
    batch, seq_rows, embed = inputs.shape
    run = pl.kernel(
        functools.partial(_sc_body_ga, seq_rows),
        out_type=jax.ShapeDtypeStruct(inputs.shape, jnp.float32),
        mesh=plsc.VectorSubcoreMesh(core_axis_name="c", subcore_axis_name="s"),
        compiler_params=pltpu.CompilerParams(use_tc_tiling_on_sc=True),
        scratch_types=[
            pltpu.VMEM((2, 2, _GT, _EMBED), jnp.float32),
            pltpu.VMEM((16,), jnp.int32),
            pltpu.SemaphoreType.DMA,
            pltpu.SemaphoreType.DMA,
            pltpu.SemaphoreType.DMA,
            pltpu.SemaphoreType.DMA,
            pltpu.SemaphoreType.DMA,
        ],
    )
    return run(inputs, pos_table)


def _sc_body(seq_rows, in_hbm, tab_hbm, out_hbm,
             in_v, out_v, tab_v, s_in0, s_in1, s_out0, s_out1, s_tab0, s_tab1):
    wid = lax.axis_index("s") * _SC_CORES + lax.axis_index("c")
    rows_per_w = seq_rows // _NW
    nchunks = rows_per_w // _T
    base_row = wid * rows_per_w
    s_in = (s_in0, s_in1)
    s_out = (s_out0, s_out1)
    s_tab = (s_tab0, s_tab1)

    def start_in(cc, b):
        p = b % 2
        row0 = base_row + cc * _T
        return pltpu.make_async_copy(
            in_hbm.at[b, pl.ds(row0, _T), :], in_v.at[p], s_in[p])

    def start_out(cc, b):
        p = b % 2
        row0 = base_row + cc * _T
        return pltpu.make_async_copy(
            out_v.at[p], out_hbm.at[b, pl.ds(row0, _T), :], s_out[p])

    def start_tab(cc, parity):
        row0 = base_row + cc * _T
        return pltpu.make_async_copy(
            tab_hbm.at[pl.ds(row0, _T), :], tab_v.at[parity], s_tab[parity])

    start_tab(0, 0).start()
    start_tab(1, 1).start()
    start_in(0, 0).start()
    start_in(0, 1).start()

    def body(k):
        for dc in range(2):
            cc = k + dc
            for b in range(_BATCH):
                p = b % 2
                if b >= 2:
                    start_out(cc, b - 2).wait()
                elif dc == 1:
                    start_out(cc - 1, b + 2).wait()
                else:
                    @pl.when(k > 0)
                    def _():
                        start_out(cc - 1, b + 2).wait()
                start_in(cc, b).wait()
                if b == 0:
                    start_tab(cc, dc).wait()

                def row_body(r, c2):
                    def col_body(j, c3):
                        for u in range(_U):
                            sl = pl.ds((j * _U + u) * 16, 16)
                            out_v[p, r, sl] = in_v[p, r, sl] + tab_v[dc, r, sl]
                        return c3
                    lax.fori_loop(0, _EMBED // (16 * _U), col_body, 0)
                    return c2

                lax.fori_loop(0, _T, row_body, 0)
                start_out(cc, b).start()
                if b < 2:
                    start_in(cc, b + 2).start()
                elif dc == 0:
                    start_in(cc + 1, b - 2).start()
                else:
                    @pl.when(k < nchunks - 2)
                    def _():
                        start_in(cc + 1, b - 2).start()
                if b == _BATCH - 1:
                    @pl.when(cc + 2 < nchunks)
                    def _():
                        start_tab(cc + 2, dc).start()

    pl.loop(0, nchunks, step=2)(body)
    start_out(nchunks - 1, 2).wait()
    start_out(nchunks - 1, 3).wait()


def _sc_add(inputs, pos_table):
    batch, seq_rows, embed = inputs.shape
    run = pl.kernel(
        functools.partial(_sc_body, seq_rows),
        out_type=jax.ShapeDtypeStruct(inputs.shape, jnp.float32),
        mesh=plsc.VectorSubcoreMesh(core_axis_name="c", subcore_axis_name="s"),
        compiler_params=pltpu.CompilerParams(use_tc_tiling_on_sc=True),
        scratch_types=[
            pltpu.VMEM((2, _T, _EMBED), jnp.float32),
            pltpu.VMEM((2, _T, _EMBED), jnp.float32),
            pltpu.VMEM((2, _T, _EMBED), jnp.float32),
            pltpu.SemaphoreType.DMA,
            pltpu.SemaphoreType.DMA,
            pltpu.SemaphoreType.DMA,
            pltpu.SemaphoreType.DMA,
            pltpu.SemaphoreType.DMA,
            pltpu.SemaphoreType.DMA,
        ],
    )
    return run(inputs, pos_table)


def _tc_add_region(inputs, pos_table, n_rows):
    batch, seq_len, embed = inputs.shape
    return pl.pallas_call(
        _tc_add_body,
        grid=(n_rows // _TC_BLK,),
        in_specs=[
            pl.BlockSpec((batch, _TC_BLK, embed), lambda i: (0, i, 0)),
            pl.BlockSpec((_TC_BLK, embed), lambda i: (i, 0)),
        ],
        out_specs=pl.BlockSpec((batch, _TC_BLK, embed), lambda i: (0, i, 0)),
        out_shape=jax.ShapeDtypeStruct((batch, n_rows, embed), inputs.dtype),
    )(inputs, pos_table)


def _sc_body_ga_region(row_base, n_rows, in_hbm, tab_hbm, out_hbm,
                       in_v, idx_v, s_ld0, s_ld1, s_ga, s_st0, s_st1):
    wid = lax.axis_index("s") * _SC_CORES + lax.axis_index("c")
    rows_per_w = n_rows // _NW
    nchunks = rows_per_w // _GT
    nsteps = nchunks * 2
    out_base = wid * rows_per_w
    in_base = row_base + out_base
    s_ld = (s_ld0, s_ld1)
    s_st = (s_st0, s_st1)

    def start_load(t, d, i):
        b = 2 * d + i
        return pltpu.make_async_copy(
            in_hbm.at[b, pl.ds(in_base + (t // 2) * _GT, _GT), :],
            in_v.at[d, i], s_ld[d])

    def start_store(t, d, i):
        b = 2 * d + i
        return pltpu.make_async_copy(
            in_v.at[d, i],
            out_hbm.at[b, pl.ds(out_base + (t // 2) * _GT, _GT), :], s_st[d])

    def ga_descr(d, i):
        return pltpu.make_async_copy(tab_hbm.at[idx_v], in_v.at[d, i], s_ga)

    for i in range(2):
        start_load(0, 0, i).start()
    lanes = lax.broadcasted_iota(jnp.int32, (16,), 0)

    def body(k):
        for d in range(2):
            t = k + d
            for i in range(2):
                start_load(t, d, i).wait()
            if d == 0:
                idx_v[...] = in_base + (t // 2) * _GT + lanes
            for i in range(2):
                pltpu.async_copy(tab_hbm.at[idx_v], in_v.at[d, i], s_ga,
                                 add=True)
            for i in range(2):
                ga_descr(d, i).wait()
            for i in range(2):
                start_store(t, d, i).start()
            if d == 1:
                for i in range(2):
                    start_store(t - 1, 0, i).wait()
            else:
                @pl.when(k > 0)
                def _():
                    for i in range(2):
                        start_store(t - 1, 1, i).wait()
            if d == 0:
                for i in range(2):
                    start_load(t + 1, 1, i).start()
            else:
                @pl.when(k < nsteps - 2)
                def _():
                    for i in range(2):
                        start_load(t + 1, 0, i).start()

    pl.loop(0, nsteps, step=2)(body)
    for i in range(2):
        start_store(nsteps - 1, 1, i).wait()


def _sc_add_ga_region(inputs, pos_table, row_base, n_rows):
    batch, seq_rows, embed = inputs.shape
    run = pl.kernel(
        functools.partial(_sc_body_ga_region, row_base, n_rows),
        out_type=jax.ShapeDtypeStruct((batch, n_rows, embed), jnp.float32),
        mesh=plsc.VectorSubcoreMesh(core_axis_name="c", subcore_axis_name="s"),
        compiler_params=pltpu.CompilerParams(use_tc_tiling_on_sc=True),
        scratch_types=[
            pltpu.VMEM((2, 2, _GT, _EMBED), jnp.float32),
            pltpu.VMEM((16,), jnp.int32),
            pltpu.SemaphoreType.DMA,
            pltpu.SemaphoreType.DMA,
            pltpu.SemaphoreType.DMA,
            pltpu.SemaphoreType.DMA,
            pltpu.SemaphoreType.DMA,
        ],
    )
    return run(inputs, pos_table)


_TC_BLK2 = 1024


def _tc_add_body2(in_ref, tab_ref, out_ref):
    out_ref[...] = in_ref[...] + tab_ref[...][None, :, :]


def _tc_add_bsplit(inputs, pos_table):
    batch, seq_len, embed = inputs.shape
    return pl.pallas_call(
        _tc_add_body2,
        grid=(seq_len // _TC_BLK2, batch),
        in_specs=[
            pl.BlockSpec((1, _TC_BLK2, embed), lambda i, b: (b, i, 0)),
            pl.BlockSpec((_TC_BLK2, embed), lambda i, b: (i, 0)),
        ],
        out_specs=pl.BlockSpec((1, _TC_BLK2, embed), lambda i, b: (b, i, 0)),
        out_shape=jax.ShapeDtypeStruct((batch, seq_len, embed), inputs.dtype),
    )(inputs, pos_table)


def kernel(inputs, pos_table):
    return _tc_add(inputs, pos_table)


# speedup vs baseline: 1.0709x; 1.0709x over previous
import jax
from jax.experimental import pallas as pl

_BLK = 512


def _add_kernel(in_ref, tab_ref, out_ref):
    out_ref[...] = in_ref[...] + tab_ref[...][None, :, :]


def kernel(inputs, pos_table):
    batch, seq_len, embed = inputs.shape
    return pl.pallas_call(
        _add_kernel,
        grid=(seq_len // _BLK,),
        in_specs=[
            pl.BlockSpec((batch, _BLK, embed), lambda i: (0, i, 0)),
            pl.BlockSpec((_BLK, embed), lambda i: (i, 0)),
        ],
        out_specs=pl.BlockSpec((batch, _BLK, embed), lambda i: (0, i, 0)),
        out_shape=jax.ShapeDtypeStruct((batch, seq_len, embed), inputs.dtype),
    )(inputs, pos_table)
